# baseline (speedup 1.0000x reference)
import jax
import jax.numpy as jnp
from jax.experimental import pallas as pl
from jax.experimental.pallas import tpu as pltpu


def _add_kernel(x_ref, t_ref, o_ref):
    o_ref[...] = x_ref[...] + t_ref[...]


def kernel(x, table):
    B, L, D = x.shape
    BL = 512
    BD = 512
    return pl.pallas_call(
        _add_kernel,
        grid=(L // BL, D // BD),
        in_specs=[
            pl.BlockSpec((B, BL, BD), lambda i, j: (0, i, j)),
            pl.BlockSpec((BL, BD), lambda i, j: (i, j)),
        ],
        out_specs=pl.BlockSpec((B, BL, BD), lambda i, j: (0, i, j)),
        out_shape=jax.ShapeDtypeStruct(x.shape, x.dtype),
        compiler_params=pltpu.CompilerParams(
            dimension_semantics=("parallel", "parallel"),
        ),
    )(x, table)


# speedup vs baseline: 1.0104x; 1.0104x over previous
import jax
import jax.numpy as jnp
from jax.experimental import pallas as pl


def _add_kernel(x_ref, t_ref, o_ref):
    o_ref[...] = x_ref[...] + t_ref[...]


def kernel(x, table):
    B, L, D = x.shape
    BL = 512
    return pl.pallas_call(
        _add_kernel,
        grid=(L // BL,),
        in_specs=[
            pl.BlockSpec((B, BL, D), lambda i: (0, i, 0)),
            pl.BlockSpec((BL, D), lambda i: (i, 0)),
        ],
        out_specs=pl.BlockSpec((B, BL, D), lambda i: (0, i, 0)),
        out_shape=jax.ShapeDtypeStruct(x.shape, x.dtype),
    )(x, table)
